# Initial kernel scaffold; baseline (speedup 1.0000x reference)
import jax
import jax.numpy as jnp
from jax.experimental import pallas as pl


def kernel(rpn_cls_prob, rpn_bbox_pred, anchors, img_size):
    raise NotImplementedError("write your pallas kernel here")


# speedup vs baseline: 10.6821x; 10.6821x over previous
import jax
import jax.numpy as jnp
from jax.experimental import pallas as pl
from jax.experimental.pallas import tpu as pltpu

F32 = jnp.float32

N_RAW = 22500
N_PAD = 22528
PRE = 12000
P_PAD = 12288
POST = 2000
THRESH = 0.7

I_BLK = 1024
J_CH = 1024
P_BLK = 512
J2 = 2048

_HI = jax.lax.Precision.HIGHEST


def _sort_kernel(s_col, s_row, vals_t, img, out, rank_scr, sorted_scr):
    def rank_blk(bi, carry):
        i0 = bi * I_BLK
        si = s_row[:, pl.ds(i0, I_BLK)]
        ii = jax.lax.broadcasted_iota(jnp.int32, (1, I_BLK), 1) + i0
        def jblk(bj, cnt):
            j0 = bj * J_CH
            sj = s_col[pl.ds(j0, J_CH), :]
            ij = jax.lax.broadcasted_iota(jnp.int32, (J_CH, 1), 0) + j0
            before = (sj > si) | ((sj == si) & (ij < ii))
            return cnt + jnp.sum(jnp.where(before, 1.0, 0.0).astype(F32),
                                 axis=0, keepdims=True)
        cnt = jax.lax.fori_loop(0, N_PAD // J_CH, jblk,
                                jnp.zeros((1, I_BLK), F32))
        rank_scr[:, pl.ds(i0, I_BLK)] = cnt
        return carry
    jax.lax.fori_loop(0, N_PAD // I_BLK, rank_blk, 0)

    def pblk(pb, carry):
        p0 = pb * P_BLK
        p_col = (jax.lax.broadcasted_iota(jnp.int32, (P_BLK, 1), 0)
                 + p0).astype(F32)
        def jblk2(jb, acc):
            j0 = jb * J2
            rr = rank_scr[:, pl.ds(j0, J2)]
            oh = jnp.where(rr == p_col, 1.0, 0.0).astype(F32)
            vj = vals_t[:, pl.ds(j0, J2)]
            return acc + jax.lax.dot_general(
                vj, oh, (((1,), (1,)), ((), ())),
                preferred_element_type=F32, precision=_HI)
        acc = jax.lax.fori_loop(0, N_PAD // J2, jblk2,
                                jnp.zeros((8, P_BLK), F32))
        sorted_scr[:, pl.ds(p0, P_BLK)] = acc
        return carry
    jax.lax.fori_loop(0, P_PAD // P_BLK, pblk, 0)

    sv = sorted_scr[:, :]
    a0, a1, a2, a3 = sv[0:1, :], sv[1:2, :], sv[2:3, :], sv[3:4, :]
    d0, d1, d2, d3 = sv[4:5, :], sv[5:6, :], sv[6:7, :], sv[7:8, :]
    w = a2 - a0 + 1.0
    h = a3 - a1 + 1.0
    cx = a0 + 0.5 * w
    cy = a1 + 0.5 * h
    pcx = d0 * w + cx
    pcy = d1 * h + cy
    pw = jnp.exp(d2) * w
    ph = jnp.exp(d3) * h
    x1 = pcx - 0.5 * pw
    y1 = pcy - 0.5 * ph
    x2 = pcx + 0.5 * pw - 1.0
    y2 = pcy + 0.5 * ph - 1.0
    m = img[0, 0] - 1.0
    x1 = jnp.clip(x1, 0.0, m)
    y1 = jnp.clip(y1, 0.0, m)
    x2 = jnp.clip(x2, 0.0, m)
    y2 = jnp.clip(y2, 0.0, m)
    out[0:1, :] = x1
    out[1:2, :] = y1
    out[2:3, :] = x2
    out[3:4, :] = y2
    out[4:5, :] = jnp.maximum(x2 - x1, 0.0) * jnp.maximum(y2 - y1, 0.0)
    out[5:8, :] = jnp.zeros((3, P_PAD), F32)


def _nms_kernel(x1, y1, x2, y2, area, prop, rois, sup_scr):
    R, C = P_PAD // 128, 128
    flat_f = (jax.lax.broadcasted_iota(jnp.int32, (R, C), 0) * 128
              + jax.lax.broadcasted_iota(jnp.int32, (R, C), 1)).astype(F32)
    sup_scr[:, :] = jnp.where(flat_f >= float(PRE), 1.0, 0.0)
    rois[:, :] = jnp.zeros((POST, 4), F32)

    X1, Y1, X2, Y2, AR = x1[:, :], y1[:, :], x2[:, :], y2[:, :], area[:, :]
    lane = jax.lax.broadcasted_iota(jnp.int32, (1, C), 1)

    def cond(st):
        count, done = st
        return jnp.logical_and(jnp.logical_not(done), count < POST)

    def body(st):
        count, done = st
        sup = sup_scr[:, :]
        nxt_f = jnp.min(jnp.where(sup == 0.0, flat_f, 3.0e7))
        none_left = nxt_f >= float(PRE)

        @pl.when(jnp.logical_not(none_left))
        def _():
            nxt = nxt_f.astype(jnp.int32)
            r = nxt // 128
            c = nxt - r * 128
            def ext(ref):
                slab = ref[pl.ds(r, 1), :]
                return jnp.sum(jnp.where(lane == c, slab, 0.0))
            bx1, by1, bx2, by2, ba = ext(x1), ext(y1), ext(x2), ext(y2), ext(area)
            rois[pl.ds(count, 1), :] = prop[pl.ds(nxt, 1), :]
            xx1 = jnp.maximum(bx1, X1)
            yy1 = jnp.maximum(by1, Y1)
            xx2 = jnp.minimum(bx2, X2)
            yy2 = jnp.minimum(by2, Y2)
            inter = jnp.maximum(xx2 - xx1, 0.0) * jnp.maximum(yy2 - yy1, 0.0)
            union = ba + AR - inter
            iou = jnp.where(union > 0.0, inter / union, 0.0)
            sup_scr[:, :] = jnp.where(
                jnp.logical_or(iou > THRESH, flat_f == nxt_f), 1.0, sup)

        return (count + jnp.where(none_left, 0, 1).astype(jnp.int32),
                none_left)

    jax.lax.while_loop(cond, body, (jnp.int32(0), jnp.bool_(False)))


@jax.jit
def kernel(rpn_cls_prob, rpn_bbox_pred, anchors, img_size):
    scores = rpn_cls_prob[..., 1].reshape(-1)
    deltas = rpn_bbox_pred.reshape(-1, 4)

    pad = N_PAD - N_RAW
    s_flat = jnp.concatenate([scores, jnp.full((pad,), -1.0, F32)])
    av = jnp.concatenate([anchors, deltas], axis=1)
    vals_t = jnp.pad(av, ((0, pad), (0, 0))).T
    img = (jnp.asarray(img_size, F32)).reshape(1, 1)

    out = pl.pallas_call(
        _sort_kernel,
        out_shape=jax.ShapeDtypeStruct((8, P_PAD), F32),
        scratch_shapes=[
            pltpu.VMEM((1, N_PAD), F32),
            pltpu.VMEM((8, P_PAD), F32),
        ],
    )(s_flat.reshape(N_PAD, 1), s_flat.reshape(1, N_PAD), vals_t, img)

    x1, y1, x2, y2, area = out[0], out[1], out[2], out[3], out[4]
    prop = jnp.stack([x1, y1, x2, y2], axis=1)

    rois = pl.pallas_call(
        _nms_kernel,
        out_shape=jax.ShapeDtypeStruct((POST, 4), F32),
        scratch_shapes=[pltpu.VMEM((P_PAD // 128, 128), F32)],
    )(x1.reshape(P_PAD // 128, 128), y1.reshape(P_PAD // 128, 128),
      x2.reshape(P_PAD // 128, 128), y2.reshape(P_PAD // 128, 128),
      area.reshape(P_PAD // 128, 128), prop)
    return rois


# speedup vs baseline: 12.1333x; 1.1358x over previous
import jax
import jax.numpy as jnp
from jax.experimental import pallas as pl
from jax.experimental.pallas import tpu as pltpu

F32 = jnp.float32

N_RAW = 22500
N_PAD = 22528
PRE = 12000
P_PAD = 12288
POST = 2000
THRESH = 0.7

I_BLK = 1024
J_CH = 1024
P_BLK = 512
J2 = 2048

_HI = jax.lax.Precision.HIGHEST


def _sort_kernel(s_col, s_row, vals_t, img, out, rank_scr, sorted_scr):
    NB = N_PAD // J_CH
    def rank_blk(bi, carry):
        i0 = bi * I_BLK
        si = s_row[:, pl.ds(i0, I_BLK)]
        def geq_blk(bj, cnt):
            sj = s_col[pl.ds(bj * J_CH, J_CH), :]
            return cnt + jnp.sum(jnp.where(sj >= si, 1.0, 0.0).astype(F32),
                                 axis=0, keepdims=True)
        def gt_blk(bj, cnt):
            sj = s_col[pl.ds(bj * J_CH, J_CH), :]
            return cnt + jnp.sum(jnp.where(sj > si, 1.0, 0.0).astype(F32),
                                 axis=0, keepdims=True)
        cnt = jax.lax.fori_loop(0, bi, geq_blk,
                                jnp.zeros((1, I_BLK), F32))
        cnt = jax.lax.fori_loop(bi + 1, NB, gt_blk, cnt)
        sj = s_col[pl.ds(i0, J_CH), :]
        ij = jax.lax.broadcasted_iota(jnp.int32, (J_CH, 1), 0)
        ii = jax.lax.broadcasted_iota(jnp.int32, (1, I_BLK), 1)
        before = (sj > si) | ((sj == si) & (ij < ii))
        cnt = cnt + jnp.sum(jnp.where(before, 1.0, 0.0).astype(F32),
                            axis=0, keepdims=True)
        rank_scr[:, pl.ds(i0, I_BLK)] = cnt
        return carry
    jax.lax.fori_loop(0, N_PAD // I_BLK, rank_blk, 0)

    def pblk(pb, carry):
        p0 = pb * P_BLK
        p_col = (jax.lax.broadcasted_iota(jnp.int32, (P_BLK, 1), 0)
                 + p0).astype(F32)
        def jblk2(jb, acc):
            j0 = jb * J2
            rr = rank_scr[:, pl.ds(j0, J2)]
            oh = jnp.where(rr == p_col, 1.0, 0.0).astype(F32)
            vj = vals_t[:, pl.ds(j0, J2)]
            return acc + jax.lax.dot_general(
                vj, oh, (((1,), (1,)), ((), ())),
                preferred_element_type=F32, precision=_HI)
        acc = jax.lax.fori_loop(0, N_PAD // J2, jblk2,
                                jnp.zeros((8, P_BLK), F32))
        sorted_scr[:, pl.ds(p0, P_BLK)] = acc
        return carry
    jax.lax.fori_loop(0, P_PAD // P_BLK, pblk, 0)

    sv = sorted_scr[:, :]
    a0, a1, a2, a3 = sv[0:1, :], sv[1:2, :], sv[2:3, :], sv[3:4, :]
    d0, d1, d2, d3 = sv[4:5, :], sv[5:6, :], sv[6:7, :], sv[7:8, :]
    w = a2 - a0 + 1.0
    h = a3 - a1 + 1.0
    cx = a0 + 0.5 * w
    cy = a1 + 0.5 * h
    pcx = d0 * w + cx
    pcy = d1 * h + cy
    pw = jnp.exp(d2) * w
    ph = jnp.exp(d3) * h
    x1 = pcx - 0.5 * pw
    y1 = pcy - 0.5 * ph
    x2 = pcx + 0.5 * pw - 1.0
    y2 = pcy + 0.5 * ph - 1.0
    m = img[0, 0] - 1.0
    x1 = jnp.clip(x1, 0.0, m)
    y1 = jnp.clip(y1, 0.0, m)
    x2 = jnp.clip(x2, 0.0, m)
    y2 = jnp.clip(y2, 0.0, m)
    out[0:1, :] = x1
    out[1:2, :] = y1
    out[2:3, :] = x2
    out[3:4, :] = y2
    out[4:5, :] = jnp.maximum(x2 - x1, 0.0) * jnp.maximum(y2 - y1, 0.0)
    out[5:8, :] = jnp.zeros((3, P_PAD), F32)


def _nms_kernel(x1, y1, x2, y2, area, prop, rois, sup_scr, nxt_scr):
    R, C = P_PAD // 128, 128
    W = 8
    flat_f = (jax.lax.broadcasted_iota(jnp.int32, (R, C), 0) * 128
              + jax.lax.broadcasted_iota(jnp.int32, (R, C), 1)).astype(F32)
    sup_scr[:, :] = jnp.where(flat_f >= float(PRE), 1.0, 0.0)
    rois[:, :] = jnp.zeros((POST, 4), F32)

    X1, Y1, X2, Y2, AR = x1[:, :], y1[:, :], x2[:, :], y2[:, :], area[:, :]
    win_base = (jax.lax.broadcasted_iota(jnp.int32, (W, C), 0) * 128
                + jax.lax.broadcasted_iota(jnp.int32, (W, C), 1)).astype(F32)
    SENT = 3.0e7

    def cond(st):
        count, done, _ = st
        return jnp.logical_and(jnp.logical_not(done), count < POST)

    def body(st):
        count, done, cur_row = st
        r0 = jnp.minimum(cur_row, R - W)
        win_sup = sup_scr[pl.ds(r0, W), :]
        win_idx = win_base + (r0 * 128).astype(F32)
        nxt_scr[0, 0] = jnp.min(jnp.where(win_sup == 0.0, win_idx, SENT))

        @pl.when(nxt_scr[0, 0] >= SENT)
        def _():
            sup = sup_scr[:, :]
            nxt_scr[0, 0] = jnp.min(jnp.where(sup == 0.0, flat_f, SENT))

        nxt_f = nxt_scr[0, 0]
        none_left = nxt_f >= float(PRE)

        @pl.when(jnp.logical_not(none_left))
        def _():
            nxt = nxt_f.astype(jnp.int32)
            row4 = prop[pl.ds(nxt, 1), :]
            rois[pl.ds(count, 1), :] = row4
            bx1 = row4[:, 0:1]
            by1 = row4[:, 1:2]
            bx2 = row4[:, 2:3]
            by2 = row4[:, 3:4]
            ba = (jnp.maximum(bx2 - bx1, 0.0)
                  * jnp.maximum(by2 - by1, 0.0))
            xx1 = jnp.maximum(bx1, X1)
            yy1 = jnp.maximum(by1, Y1)
            xx2 = jnp.minimum(bx2, X2)
            yy2 = jnp.minimum(by2, Y2)
            inter = jnp.maximum(xx2 - xx1, 0.0) * jnp.maximum(yy2 - yy1, 0.0)
            union = ba + AR - inter
            iou = jnp.where(union > 0.0, inter / union, 0.0)
            sup_scr[:, :] = jnp.where(
                jnp.logical_or(iou > THRESH, flat_f == nxt_f), 1.0,
                sup_scr[:, :])

        new_row = jnp.where(none_left, cur_row,
                            nxt_f.astype(jnp.int32) // 128)
        return (count + jnp.where(none_left, 0, 1).astype(jnp.int32),
                none_left, new_row)

    jax.lax.while_loop(cond, body,
                       (jnp.int32(0), jnp.bool_(False), jnp.int32(0)))


@jax.jit
def kernel(rpn_cls_prob, rpn_bbox_pred, anchors, img_size):
    scores = rpn_cls_prob[..., 1].reshape(-1)
    deltas = rpn_bbox_pred.reshape(-1, 4)

    pad = N_PAD - N_RAW
    s_flat = jnp.concatenate([scores, jnp.full((pad,), -1.0, F32)])
    av = jnp.concatenate([anchors, deltas], axis=1)
    vals_t = jnp.pad(av, ((0, pad), (0, 0))).T
    img = (jnp.asarray(img_size, F32)).reshape(1, 1)

    out = pl.pallas_call(
        _sort_kernel,
        out_shape=jax.ShapeDtypeStruct((8, P_PAD), F32),
        scratch_shapes=[
            pltpu.VMEM((1, N_PAD), F32),
            pltpu.VMEM((8, P_PAD), F32),
        ],
    )(s_flat.reshape(N_PAD, 1), s_flat.reshape(1, N_PAD), vals_t, img)

    x1, y1, x2, y2, area = out[0], out[1], out[2], out[3], out[4]
    prop = jnp.stack([x1, y1, x2, y2], axis=1)

    rois = pl.pallas_call(
        _nms_kernel,
        out_shape=jax.ShapeDtypeStruct((POST, 4), F32),
        scratch_shapes=[pltpu.VMEM((P_PAD // 128, 128), F32),
                        pltpu.SMEM((1, 1), F32)],
    )(x1.reshape(P_PAD // 128, 128), y1.reshape(P_PAD // 128, 128),
      x2.reshape(P_PAD // 128, 128), y2.reshape(P_PAD // 128, 128),
      area.reshape(P_PAD // 128, 128), prop)
    return rois


# speedup vs baseline: 21.2140x; 1.7484x over previous
import jax
import jax.numpy as jnp
from jax.experimental import pallas as pl
from jax.experimental.pallas import tpu as pltpu
from jax.experimental.pallas import tpu_sc as plsc

F32 = jnp.float32

N_RAW = 22500
N_PAD = 22528
PRE = 12000
P_PAD = 12288
POST = 2000
THRESH = 0.7

I_BLK = 1024
J_CH = 1024

SC_NC = 2
SC_NS = 16
SC_NW = SC_NC * SC_NS
B_PER_W = N_PAD // SC_NW
SC_CHUNK = 88
SC_NCH = B_PER_W // SC_CHUNK
ROW_W = 128


def _rank_decode_kernel(s_col, s_row, anch_t, delt_t, img, rank_out, boxes_t):
    NB = N_PAD // J_CH
    def rank_blk(bi, carry):
        i0 = bi * I_BLK
        si = s_row[:, pl.ds(i0, I_BLK)]
        def geq_blk(bj, cnt):
            sj = s_col[pl.ds(bj * J_CH, J_CH), :]
            return cnt + jnp.sum(jnp.where(sj >= si, 1.0, 0.0).astype(F32),
                                 axis=0, keepdims=True)
        def gt_blk(bj, cnt):
            sj = s_col[pl.ds(bj * J_CH, J_CH), :]
            return cnt + jnp.sum(jnp.where(sj > si, 1.0, 0.0).astype(F32),
                                 axis=0, keepdims=True)
        cnt = jax.lax.fori_loop(0, bi, geq_blk,
                                jnp.zeros((1, I_BLK), F32))
        cnt = jax.lax.fori_loop(bi + 1, NB, gt_blk, cnt)
        sj = s_col[pl.ds(i0, J_CH), :]
        ij = jax.lax.broadcasted_iota(jnp.int32, (J_CH, 1), 0)
        ii = jax.lax.broadcasted_iota(jnp.int32, (1, I_BLK), 1)
        before = (sj > si) | ((sj == si) & (ij < ii))
        cnt = cnt + jnp.sum(jnp.where(before, 1.0, 0.0).astype(F32),
                            axis=0, keepdims=True)
        rank_out[:, pl.ds(i0, I_BLK)] = cnt.astype(jnp.int32)
        return carry
    jax.lax.fori_loop(0, N_PAD // I_BLK, rank_blk, 0)

    a0, a1, a2, a3 = (anch_t[0:1, :], anch_t[1:2, :],
                      anch_t[2:3, :], anch_t[3:4, :])
    d0, d1, d2, d3 = (delt_t[0:1, :], delt_t[1:2, :],
                      delt_t[2:3, :], delt_t[3:4, :])
    w = a2 - a0 + 1.0
    h = a3 - a1 + 1.0
    cx = a0 + 0.5 * w
    cy = a1 + 0.5 * h
    pcx = d0 * w + cx
    pcy = d1 * h + cy
    pw = jnp.exp(d2) * w
    ph = jnp.exp(d3) * h
    x1 = pcx - 0.5 * pw
    y1 = pcy - 0.5 * ph
    x2 = pcx + 0.5 * pw - 1.0
    y2 = pcy + 0.5 * ph - 1.0
    m = img[0, 0] - 1.0
    x1 = jnp.clip(x1, 0.0, m)
    y1 = jnp.clip(y1, 0.0, m)
    x2 = jnp.clip(x2, 0.0, m)
    y2 = jnp.clip(y2, 0.0, m)
    boxes_t[0:1, :] = x1
    boxes_t[1:2, :] = y1
    boxes_t[2:3, :] = x2
    boxes_t[3:4, :] = y2
    boxes_t[4:5, :] = jnp.maximum(x2 - x1, 0.0) * jnp.maximum(y2 - y1, 0.0)
    boxes_t[5:8, :] = jnp.zeros((3, N_PAD), F32)


def _sc_scatter_kernel(rows_hbm, rank_hbm, out_hbm, idx_v, rows_v, sem):
    wid = jax.lax.axis_index("s") * SC_NC + jax.lax.axis_index("c")
    base = wid * B_PER_W
    for k in range(SC_NCH):
        off = base + k * SC_CHUNK
        pltpu.sync_copy(rank_hbm.at[pl.ds(off, SC_CHUNK)], idx_v)
        pltpu.sync_copy(rows_hbm.at[pl.ds(off, SC_CHUNK)], rows_v)
        pltpu.async_copy(rows_v, out_hbm.at[idx_v], sem).wait()


def _nms_kernel(x1, y1, x2, y2, area, prop, rois, sup_scr, nxt_scr):
    R, C = P_PAD // 128, 128
    W = 8
    flat_f = (jax.lax.broadcasted_iota(jnp.int32, (R, C), 0) * 128
              + jax.lax.broadcasted_iota(jnp.int32, (R, C), 1)).astype(F32)
    sup_scr[:, :] = jnp.where(flat_f >= float(PRE), 1.0, 0.0)
    rois[:, :] = jnp.zeros((POST, 4), F32)

    X1, Y1, X2, Y2, AR = x1[:, :], y1[:, :], x2[:, :], y2[:, :], area[:, :]
    win_base = (jax.lax.broadcasted_iota(jnp.int32, (W, C), 0) * 128
                + jax.lax.broadcasted_iota(jnp.int32, (W, C), 1)).astype(F32)
    SENT = 3.0e7

    def cond(st):
        count, done, _ = st
        return jnp.logical_and(jnp.logical_not(done), count < POST)

    def body(st):
        count, done, cur_row = st
        r0 = jnp.minimum(cur_row, R - W)
        win_sup = sup_scr[pl.ds(r0, W), :]
        win_idx = win_base + (r0 * 128).astype(F32)
        nxt_scr[0, 0] = jnp.min(jnp.where(win_sup == 0.0, win_idx, SENT))

        @pl.when(nxt_scr[0, 0] >= SENT)
        def _():
            sup = sup_scr[:, :]
            nxt_scr[0, 0] = jnp.min(jnp.where(sup == 0.0, flat_f, SENT))

        nxt_f = nxt_scr[0, 0]
        none_left = nxt_f >= float(PRE)

        @pl.when(jnp.logical_not(none_left))
        def _():
            nxt = nxt_f.astype(jnp.int32)
            row4 = prop[pl.ds(nxt, 1), :]
            rois[pl.ds(count, 1), :] = row4
            bx1 = row4[:, 0:1]
            by1 = row4[:, 1:2]
            bx2 = row4[:, 2:3]
            by2 = row4[:, 3:4]
            ba = (jnp.maximum(bx2 - bx1, 0.0)
                  * jnp.maximum(by2 - by1, 0.0))
            xx1 = jnp.maximum(bx1, X1)
            yy1 = jnp.maximum(by1, Y1)
            xx2 = jnp.minimum(bx2, X2)
            yy2 = jnp.minimum(by2, Y2)
            inter = jnp.maximum(xx2 - xx1, 0.0) * jnp.maximum(yy2 - yy1, 0.0)
            union = ba + AR - inter
            iou = jnp.where(union > 0.0, inter / union, 0.0)
            sup_scr[:, :] = jnp.where(
                jnp.logical_or(iou > THRESH, flat_f == nxt_f), 1.0,
                sup_scr[:, :])

        new_row = jnp.where(none_left, cur_row,
                            nxt_f.astype(jnp.int32) // 128)
        return (count + jnp.where(none_left, 0, 1).astype(jnp.int32),
                none_left, new_row)

    jax.lax.while_loop(cond, body,
                       (jnp.int32(0), jnp.bool_(False), jnp.int32(0)))


@jax.jit
def kernel(rpn_cls_prob, rpn_bbox_pred, anchors, img_size):
    scores = rpn_cls_prob[..., 1].reshape(-1)
    deltas = rpn_bbox_pred.reshape(-1, 4)

    pad = N_PAD - N_RAW
    s_flat = jnp.concatenate([scores, jnp.full((pad,), -1.0, F32)])
    anch_t = jnp.pad(anchors, ((0, pad), (0, 0))).T
    delt_t = jnp.pad(deltas, ((0, pad), (0, 0))).T
    img = (jnp.asarray(img_size, F32)).reshape(1, 1)

    rank_row, boxes_t = pl.pallas_call(
        _rank_decode_kernel,
        out_shape=(jax.ShapeDtypeStruct((1, N_PAD), jnp.int32),
                   jax.ShapeDtypeStruct((8, N_PAD), F32)),
    )(s_flat.reshape(N_PAD, 1), s_flat.reshape(1, N_PAD), anch_t, delt_t, img)

    rows = jnp.pad(boxes_t.T, ((0, 0), (0, ROW_W - 8)))
    rank1d = rank_row.reshape(N_PAD)

    mesh = plsc.VectorSubcoreMesh(core_axis_name="c", subcore_axis_name="s",
                                  num_cores=SC_NC, num_subcores=SC_NS)
    sorted_rows = pl.kernel(
        _sc_scatter_kernel,
        out_type=jax.ShapeDtypeStruct((N_PAD, ROW_W), F32),
        mesh=mesh,
        scratch_types=[
            pltpu.VMEM((SC_CHUNK,), jnp.int32),
            pltpu.VMEM((SC_CHUNK, ROW_W), F32),
            pltpu.SemaphoreType.DMA,
        ],
    )(rows, rank1d)

    prop = sorted_rows[:P_PAD, :4]
    x1g = sorted_rows[:P_PAD, 0].reshape(P_PAD // 128, 128)
    y1g = sorted_rows[:P_PAD, 1].reshape(P_PAD // 128, 128)
    x2g = sorted_rows[:P_PAD, 2].reshape(P_PAD // 128, 128)
    y2g = sorted_rows[:P_PAD, 3].reshape(P_PAD // 128, 128)
    areag = sorted_rows[:P_PAD, 4].reshape(P_PAD // 128, 128)

    rois = pl.pallas_call(
        _nms_kernel,
        out_shape=jax.ShapeDtypeStruct((POST, 4), F32),
        scratch_shapes=[pltpu.VMEM((P_PAD // 128, 128), F32),
                        pltpu.SMEM((1, 1), F32)],
    )(x1g, y1g, x2g, y2g, areag, prop)
    return rois
